# Initial kernel scaffold; baseline (speedup 1.0000x reference)
import jax
import jax.numpy as jnp
from jax.experimental import pallas as pl


def kernel(x, y):
    raise NotImplementedError("write your pallas kernel here")


# speedup vs baseline: 1.2922x; 1.2922x over previous
import functools

import jax
import jax.numpy as jnp
from jax import lax
from jax.experimental import pallas as pl
from jax.experimental.pallas import tpu as pltpu
from jax.experimental.pallas import tpu_sc as plsc

_N_PROJ = 256
_NC, _NS, _LANES = 2, 16, 16
_NW = _NC * _NS


def _proj_body(x_ref, y_ref, th_ref, xo_ref, yo_ref):
    th = th_ref[...]
    nrm = jnp.maximum(jnp.sqrt(jnp.sum(th * th, axis=1, keepdims=True)), 1e-12)
    thn = th / nrm
    dn = (((1,), (1,)), ((), ()))
    xo_ref[...] = lax.dot_general(thn, x_ref[...], dn,
                                  preferred_element_type=jnp.float32)
    yo_ref[...] = lax.dot_general(thn, y_ref[...], dn,
                                  preferred_element_type=jnp.float32)


def _project(x, y, th_t):
    n, m = x.shape[0], y.shape[0]
    return pl.pallas_call(
        _proj_body,
        out_shape=[jax.ShapeDtypeStruct((_N_PROJ, n), jnp.float32),
                   jax.ShapeDtypeStruct((_N_PROJ, m), jnp.float32)],
    )(x, y, th_t)


def _vsort_pass(buf, nvec):
    def body(i, _):
        v = buf[pl.ds(i * 16, 16)]
        buf[pl.ds(i * 16, 16)] = plsc.sort_key_val(v, v)[0]
        return 0
    lax.fori_loop(0, nvec, body, 0)


def _sort_ref(buf, n):
    _vsort_pass(buf, n // 16)
    for lev in range(5, n.bit_length()):
        def first(t, _, lev=lev):
            rb = lev - 5
            p = t >> rb
            i = t & ((1 << rb) - 1)
            ao = (p << lev) + (i << 4)
            bo = ((p + 1) << lev) - 16 - (i << 4)
            va = buf[pl.ds(ao, 16)]
            vb = lax.rev(buf[pl.ds(bo, 16)], (0,))
            buf[pl.ds(ao, 16)] = jnp.minimum(va, vb)
            buf[pl.ds(bo, 16)] = lax.rev(jnp.maximum(va, vb), (0,))
            return 0
        lax.fori_loop(0, n // 32, first, 0)
        for dl in range(lev - 2, 3, -1):
            def stage(t, _, dl=dl):
                rb = dl - 4
                g = t >> rb
                off = (t & ((1 << rb) - 1)) << 4
                i1 = (g << (dl + 1)) + off
                i2 = i1 + (1 << dl)
                va = buf[pl.ds(i1, 16)]
                vb = buf[pl.ds(i2, 16)]
                buf[pl.ds(i1, 16)] = jnp.minimum(va, vb)
                buf[pl.ds(i2, 16)] = jnp.maximum(va, vb)
                return 0
            lax.fori_loop(0, n // 32, stage, 0)
        _vsort_pass(buf, n // 16)


def _interp_ssq(xbuf, ybuf, m):
    iota = lax.iota(jnp.int32, 16)

    def body(kb, acc):
        k0 = kb * 16
        idx = k0 + iota
        vk = ybuf[pl.ds(k0, 16)]
        vkm = plsc.load_gather(ybuf, [jnp.maximum(idx - 1, 0)])
        vkp = plsc.load_gather(ybuf, [jnp.minimum(idx + 1, m - 1)])
        ye = 0.25 * vkm + 0.75 * vk
        yo = 0.75 * vk + 0.25 * vkp
        xe = plsc.load_gather(xbuf, [idx * 2])
        xo = plsc.load_gather(xbuf, [idx * 2 + 1])
        de = xe - ye
        do = xo - yo
        return acc + de * de + do * do

    return lax.fori_loop(0, m // 16, body, jnp.zeros((16,), jnp.float32))


def _make_sc(n, m, ncols):
    cpw = ncols // _NW
    mesh = plsc.VectorSubcoreMesh(core_axis_name="c", subcore_axis_name="s",
                                  num_cores=_NC, num_subcores=_NS)

    @functools.partial(
        pl.kernel,
        out_type=jax.ShapeDtypeStruct((_NW, _LANES), jnp.float32),
        mesh=mesh,
        scratch_types=[pltpu.VMEM((n,), jnp.float32),
                       pltpu.VMEM((m,), jnp.float32),
                       pltpu.VMEM((_LANES,), jnp.float32)],
        compiler_params=pltpu.CompilerParams(needs_layout_passes=False),
    )
    def sc(xp_hbm, yp_hbm, out_hbm, xbuf, ybuf, obuf):
        wid = lax.axis_index("s") * _NC + lax.axis_index("c")

        def col_body(c, acc):
            col = wid * cpw + c
            pltpu.sync_copy(xp_hbm.at[col], xbuf)
            pltpu.sync_copy(yp_hbm.at[col], ybuf)
            _sort_ref(xbuf, n)
            _sort_ref(ybuf, m)
            return acc + _interp_ssq(xbuf, ybuf, m)

        acc = lax.fori_loop(0, cpw, col_body, jnp.zeros((16,), jnp.float32))
        obuf[...] = acc
        pltpu.sync_copy(obuf, out_hbm.at[wid])

    return sc


def kernel(x, y):
    n, d = x.shape
    m = y.shape[0]
    assert n == 2 * m, "kernel specialized to N == 2*M"
    theta = jax.random.normal(jax.random.key(42), (d, _N_PROJ), dtype=x.dtype)
    th_t = theta.T
    xp, yp = _project(x, y, th_t)
    parts = _make_sc(n, m, _N_PROJ)(xp, yp)
    swd2 = jnp.sum(parts) / (n * _N_PROJ)
    return jnp.maximum(jnp.sqrt(swd2), jnp.float32(1e-8))


# speedup vs baseline: 8.5781x; 6.6386x over previous
import functools

import jax
import jax.numpy as jnp
from jax import lax
from jax.experimental import pallas as pl
from jax.experimental.pallas import tpu as pltpu
from jax.experimental.pallas import tpu_sc as plsc

_N_PROJ = 256
_NC, _NS, _LANES = 2, 16, 16
_NW = _NC * _NS


def _proj_body(x_ref, y_ref, th_ref, xo_ref, yo_ref):
    th = th_ref[...]
    nrm = jnp.maximum(jnp.sqrt(jnp.sum(th * th, axis=1, keepdims=True)), 1e-12)
    thn = th / nrm
    dn = (((1,), (1,)), ((), ()))
    xo_ref[...] = lax.dot_general(thn, x_ref[...], dn,
                                  preferred_element_type=jnp.float32)
    yo_ref[...] = lax.dot_general(thn, y_ref[...], dn,
                                  preferred_element_type=jnp.float32)


def _project(x, y, th_t):
    n, m = x.shape[0], y.shape[0]
    return pl.pallas_call(
        _proj_body,
        out_shape=[jax.ShapeDtypeStruct((_N_PROJ, n), jnp.float32),
                   jax.ShapeDtypeStruct((_N_PROJ, m), jnp.float32)],
    )(x, y, th_t)


def _vs(v):
    return plsc.sort_key_val(v, v)[0]


def _cmpx(a, b):
    return jnp.minimum(a, b), jnp.maximum(a, b)


def _aligned_net(w):
    m = len(w)
    d = m // 2
    while d >= 1:
        for base in range(0, m, 2 * d):
            for o in range(d):
                w[base + o], w[base + o + d] = _cmpx(w[base + o], w[base + o + d])
        d //= 2
    return w


def _merge_lists(c):
    r2 = len(c)
    for i in range(r2 // 2):
        j = r2 - 1 - i
        vb = lax.rev(c[j], (0,))
        lo, hi = _cmpx(c[i], vb)
        c[i] = lo
        c[j] = lax.rev(hi, (0,))
    half = r2 // 2
    if half >= 2:
        c[:half] = _aligned_net(c[:half])
        c[half:] = _aligned_net(c[half:])
    return [_vs(v) for v in c]


def _sort_block16(c):
    c = [_vs(v) for v in c]
    width = 1
    while width < 16:
        for p in range(0, 16, 2 * width):
            c[p:p + 2 * width] = _merge_lists(c[p:p + 2 * width])
        width *= 2
    return c


def _ld(buf, u):
    return buf[pl.ds(u * 16, 16)]


def _st(buf, u, v):
    buf[pl.ds(u * 16, 16)] = v


def _mega_pass(buf, nv):
    def body(t, _):
        b = t * 16
        w = [_ld(buf, b + k) for k in range(16)]
        w = _sort_block16(w)
        for k in range(16):
            _st(buf, b + k, w[k])
        return 0
    lax.fori_loop(0, nv // 16, body, 0)


def _first_pass(buf, nv, lev):
    lrv = lev - 5

    def body(t, _):
        p = t >> (lrv - 3)
        io = (t << 3) & ((1 << lrv) - 1)
        a0 = (p << (lrv + 1)) + io
        b0 = (p << (lrv + 1)) + (2 << lrv) - 1 - io
        for k in range(8):
            va = _ld(buf, a0 + k)
            vb = lax.rev(_ld(buf, b0 - k), (0,))
            lo, hi = _cmpx(va, vb)
            _st(buf, a0 + k, lo)
            _st(buf, b0 - k, lax.rev(hi, (0,)))
        return 0
    lax.fori_loop(0, nv // 16, body, 0)


def _upper_pass(buf, nv, m):
    nb = 16 // m
    lnb = nb.bit_length() - 1

    def body(t, _):
        tb = t << lnb
        base = ((tb >> 4) << (4 + (m.bit_length() - 1))) + (tb & 15)
        for q in range(nb):
            w = [_ld(buf, base + q + k * 16) for k in range(m)]
            w = _aligned_net(w)
            for k in range(m):
                _st(buf, base + q + k * 16, w[k])
        return 0
    lax.fori_loop(0, nv // 16, body, 0)


def _low_pass(buf, nv):
    def body(t, _):
        b = t * 16
        w = [_ld(buf, b + k) for k in range(16)]
        w = _aligned_net(w)
        w = [_vs(v) for v in w]
        for k in range(16):
            _st(buf, b + k, w[k])
        return 0
    lax.fori_loop(0, nv // 16, body, 0)


def _low_pass_interp(xbuf, ybuf, nv, m, acc):
    iota = lax.iota(jnp.int32, 16)
    even = (iota & 1) == 0

    def body(t, acc):
        b = t * 16
        w = [_ld(xbuf, b + k) for k in range(16)]
        w = _aligned_net(w)
        w = [_vs(v) for v in w]
        for k in range(16):
            j = (b + k) * 16 + iota
            ia = j >> 1
            ib = jnp.where(even, jnp.maximum(ia - 1, 0),
                           jnp.minimum(ia + 1, m - 1))
            ya = plsc.load_gather(ybuf, [ia])
            yb = plsc.load_gather(ybuf, [ib])
            d = w[k] - (0.75 * ya + 0.25 * yb)
            acc = acc + d * d
        return acc
    return lax.fori_loop(0, nv // 16, body, acc)


def _sort_ref(buf, n):
    nv = n // 16
    _mega_pass(buf, nv)
    for lev in range(9, n.bit_length()):
        _first_pass(buf, nv, lev)
        if lev >= 10:
            _upper_pass(buf, nv, 1 << (lev - 9))
        _low_pass(buf, nv)


def _make_sc(n, m, ncols):
    cpw = ncols // _NW
    nvx, nvy = n // 16, m // 16
    mesh = plsc.VectorSubcoreMesh(core_axis_name="c", subcore_axis_name="s",
                                  num_cores=_NC, num_subcores=_NS)

    @functools.partial(
        pl.kernel,
        out_type=jax.ShapeDtypeStruct((_NW, _LANES), jnp.float32),
        mesh=mesh,
        scratch_types=[pltpu.VMEM((n,), jnp.float32),
                       pltpu.VMEM((m,), jnp.float32),
                       pltpu.VMEM((_LANES,), jnp.float32)],
        compiler_params=pltpu.CompilerParams(needs_layout_passes=False),
    )
    def sc(xp_hbm, yp_hbm, out_hbm, xbuf, ybuf, obuf):
        wid = lax.axis_index("s") * _NC + lax.axis_index("c")

        def col_body(c, acc):
            col = wid * cpw + c
            pltpu.sync_copy(yp_hbm.at[col], ybuf)
            _sort_ref(ybuf, m)
            pltpu.sync_copy(xp_hbm.at[col], xbuf)
            _mega_pass(xbuf, nvx)
            for lev in range(9, n.bit_length()):
                _first_pass(xbuf, nvx, lev)
                if lev >= 10:
                    _upper_pass(xbuf, nvx, 1 << (lev - 9))
                if lev < n.bit_length() - 1:
                    _low_pass(xbuf, nvx)
            return _low_pass_interp(xbuf, ybuf, nvx, m, acc)

        acc = lax.fori_loop(0, cpw, col_body, jnp.zeros((16,), jnp.float32))
        obuf[...] = acc
        pltpu.sync_copy(obuf, out_hbm.at[wid])

    return sc


def kernel(x, y):
    n, d = x.shape
    m = y.shape[0]
    assert n == 2 * m, "kernel specialized to N == 2*M"
    theta = jax.random.normal(jax.random.key(42), (d, _N_PROJ), dtype=x.dtype)
    th_t = theta.T
    xp, yp = _project(x, y, th_t)
    parts = _make_sc(n, m, _N_PROJ)(xp, yp)
    swd2 = jnp.sum(parts) / (n * _N_PROJ)
    return jnp.maximum(jnp.sqrt(swd2), jnp.float32(1e-8))


# speedup vs baseline: 13.0121x; 1.5169x over previous
import functools

import jax
import jax.numpy as jnp
from jax import lax
from jax.experimental import pallas as pl
from jax.experimental.pallas import tpu as pltpu
from jax.experimental.pallas import tpu_sc as plsc

_N_PROJ = 256
_NC, _NS, _LANES = 2, 16, 16
_NW = _NC * _NS


def _proj_body(x_ref, y_ref, th_ref, xo_ref, yo_ref):
    th = th_ref[...]
    nrm = jnp.maximum(jnp.sqrt(jnp.sum(th * th, axis=1, keepdims=True)), 1e-12)
    thn = th / nrm
    dn = (((1,), (1,)), ((), ()))
    xo_ref[...] = lax.dot_general(thn, x_ref[...], dn,
                                  preferred_element_type=jnp.float32)
    yo_ref[...] = lax.dot_general(thn, y_ref[...], dn,
                                  preferred_element_type=jnp.float32)


def _project(x, y, th_t):
    n, m = x.shape[0], y.shape[0]
    return pl.pallas_call(
        _proj_body,
        out_shape=[jax.ShapeDtypeStruct((_N_PROJ, n), jnp.float32),
                   jax.ShapeDtypeStruct((_N_PROJ, m), jnp.float32)],
    )(x, y, th_t)


def _vs(v):
    return plsc.sort_key_val(v, v)[0]


def _cmpx(a, b):
    return jnp.minimum(a, b), jnp.maximum(a, b)


def _aligned_net(w):
    m = len(w)
    d = m // 2
    while d >= 1:
        for base in range(0, m, 2 * d):
            for o in range(d):
                w[base + o], w[base + o + d] = _cmpx(w[base + o], w[base + o + d])
        d //= 2
    return w


def _merge_lists(c):
    r2 = len(c)
    for i in range(r2 // 2):
        j = r2 - 1 - i
        vb = lax.rev(c[j], (0,))
        lo, hi = _cmpx(c[i], vb)
        c[i] = lo
        c[j] = lax.rev(hi, (0,))
    half = r2 // 2
    if half >= 2:
        c[:half] = _aligned_net(c[:half])
        c[half:] = _aligned_net(c[half:])
    return [_vs(v) for v in c]


def _sort_block16(c):
    c = [_vs(v) for v in c]
    width = 1
    while width < 16:
        for p in range(0, 16, 2 * width):
            c[p:p + 2 * width] = _merge_lists(c[p:p + 2 * width])
        width *= 2
    return c


def _ld(buf, u):
    return buf[pl.ds(u * 16, 16)]


def _st(buf, u, v):
    buf[pl.ds(u * 16, 16)] = v


def _mega_pass(buf, nv):
    @plsc.parallel_loop(0, nv // 16, unroll=1)
    def body(t):
        b = t * 16
        w = [_ld(buf, b + k) for k in range(16)]
        w = _sort_block16(w)
        for k in range(16):
            _st(buf, b + k, w[k])


def _first_pass(buf, nv, lev):
    lrv = lev - 5

    @plsc.parallel_loop(0, nv // 16, unroll=1)
    def body(t):
        p = t >> (lrv - 3)
        io = (t << 3) & ((1 << lrv) - 1)
        a0 = (p << (lrv + 1)) + io
        b0 = (p << (lrv + 1)) + (2 << lrv) - 1 - io
        for k in range(8):
            va = _ld(buf, a0 + k)
            vb = lax.rev(_ld(buf, b0 - k), (0,))
            lo, hi = _cmpx(va, vb)
            _st(buf, a0 + k, lo)
            _st(buf, b0 - k, lax.rev(hi, (0,)))


def _upper_pass(buf, nv, m):
    nb = 16 // m
    lnb = nb.bit_length() - 1

    @plsc.parallel_loop(0, nv // 16, unroll=1)
    def body(t):
        tb = t << lnb
        base = ((tb >> 4) << (4 + (m.bit_length() - 1))) + (tb & 15)
        for q in range(nb):
            w = [_ld(buf, base + q + k * 16) for k in range(m)]
            w = _aligned_net(w)
            for k in range(m):
                _st(buf, base + q + k * 16, w[k])


def _low_pass(buf, nv):
    @plsc.parallel_loop(0, nv // 16, unroll=1)
    def body(t):
        b = t * 16
        w = [_ld(buf, b + k) for k in range(16)]
        w = _aligned_net(w)
        w = [_vs(v) for v in w]
        for k in range(16):
            _st(buf, b + k, w[k])


def _low_pass_interp(xbuf, ybuf, nv, m, acc):
    iota = lax.iota(jnp.int32, 16)
    even = (iota & 1) == 0

    @plsc.parallel_loop(0, nv // 16, unroll=1, carry=acc)
    def body(t, acc):
        b = t * 16
        w = [_ld(xbuf, b + k) for k in range(16)]
        w = _aligned_net(w)
        w = [_vs(v) for v in w]
        for k in range(16):
            j = (b + k) * 16 + iota
            ia = j >> 1
            ib = jnp.where(even, jnp.maximum(ia - 1, 0),
                           jnp.minimum(ia + 1, m - 1))
            ya = plsc.load_gather(ybuf, [ia])
            yb = plsc.load_gather(ybuf, [ib])
            d = w[k] - (0.75 * ya + 0.25 * yb)
            acc = acc + d * d
        return acc
    return body


def _sort_ref(buf, n):
    nv = n // 16
    _mega_pass(buf, nv)
    for lev in range(9, n.bit_length()):
        _first_pass(buf, nv, lev)
        if lev >= 10:
            _upper_pass(buf, nv, 1 << (lev - 9))
        _low_pass(buf, nv)


def _make_sc(n, m, ncols):
    cpw = ncols // _NW
    nvx, nvy = n // 16, m // 16
    mesh = plsc.VectorSubcoreMesh(core_axis_name="c", subcore_axis_name="s",
                                  num_cores=_NC, num_subcores=_NS)

    @functools.partial(
        pl.kernel,
        out_type=jax.ShapeDtypeStruct((_NW, _LANES), jnp.float32),
        mesh=mesh,
        scratch_types=[pltpu.VMEM((n,), jnp.float32),
                       pltpu.VMEM((m,), jnp.float32),
                       pltpu.VMEM((_LANES,), jnp.float32)],
        compiler_params=pltpu.CompilerParams(needs_layout_passes=False),
    )
    def sc(xp_hbm, yp_hbm, out_hbm, xbuf, ybuf, obuf):
        wid = lax.axis_index("s") * _NC + lax.axis_index("c")

        def col_body(c, acc):
            col = wid * cpw + c
            pltpu.sync_copy(yp_hbm.at[col], ybuf)
            _sort_ref(ybuf, m)
            pltpu.sync_copy(xp_hbm.at[col], xbuf)
            _mega_pass(xbuf, nvx)
            for lev in range(9, n.bit_length()):
                _first_pass(xbuf, nvx, lev)
                if lev >= 10:
                    _upper_pass(xbuf, nvx, 1 << (lev - 9))
                if lev < n.bit_length() - 1:
                    _low_pass(xbuf, nvx)
            return _low_pass_interp(xbuf, ybuf, nvx, m, acc)

        acc = lax.fori_loop(0, cpw, col_body, jnp.zeros((16,), jnp.float32))
        obuf[...] = acc
        pltpu.sync_copy(obuf, out_hbm.at[wid])

    return sc


def kernel(x, y):
    n, d = x.shape
    m = y.shape[0]
    assert n == 2 * m, "kernel specialized to N == 2*M"
    theta = jax.random.normal(jax.random.key(42), (d, _N_PROJ), dtype=x.dtype)
    th_t = theta.T
    xp, yp = _project(x, y, th_t)
    parts = _make_sc(n, m, _N_PROJ)(xp, yp)
    swd2 = jnp.sum(parts) / (n * _N_PROJ)
    return jnp.maximum(jnp.sqrt(swd2), jnp.float32(1e-8))
